# baseline (speedup 1.0000x reference)
import functools

import jax
import jax.numpy as jnp
from jax import lax
from jax.experimental import pallas as pl
from jax.experimental.pallas import tpu as pltpu
from jax.experimental.pallas import tpu_sc as plsc

_NUM_TOKENS = 32768
_HIDDEN = 768
_NUM_EXPERTS = 64
_TOP_K = 8

_ROW_BLOCK = 4096

_NUM_WORKERS = 32
_ROWS_PER_W = _NUM_TOKENS // _NUM_WORKERS
_LANES = 16
_GROUPS_PER_W = _ROWS_PER_W // _LANES

_LG_WROWS = _NUM_TOKENS * _NUM_EXPERTS // 128
_LG_WROWS_W = _LG_WROWS // _NUM_WORKERS
_OUT_WROWS = _NUM_TOKENS * _TOP_K // 128
_OUT_WROWS_W = _OUT_WROWS // _NUM_WORKERS


def _tc_body(x_ref, w_ref, logits_ref, probs_ref):
    x = x_ref[...]
    w = w_ref[...]
    logits = lax.dot_general(
        x, w, (((1,), (1,)), ((), ())), preferred_element_type=jnp.float32
    )
    m = jnp.max(logits, axis=1, keepdims=True)
    e = jnp.exp(logits - m)
    probs = e / jnp.sum(e, axis=1, keepdims=True)
    logits_ref[...] = logits
    probs_ref[...] = probs


def _tc_logits_probs(x, W):
    grid = (_NUM_TOKENS // _ROW_BLOCK,)
    out_shape = jax.ShapeDtypeStruct((_NUM_TOKENS, _NUM_EXPERTS), jnp.float32)
    return pl.pallas_call(
        _tc_body,
        grid=grid,
        in_specs=[
            pl.BlockSpec((_ROW_BLOCK, _HIDDEN), lambda i: (i, 0)),
            pl.BlockSpec((_NUM_EXPERTS, _HIDDEN), lambda i: (0, 0)),
        ],
        out_specs=[
            pl.BlockSpec((_ROW_BLOCK, _NUM_EXPERTS), lambda i: (i, 0)),
            pl.BlockSpec((_ROW_BLOCK, _NUM_EXPERTS), lambda i: (i, 0)),
        ],
        out_shape=[out_shape, out_shape],
        compiler_params=pltpu.CompilerParams(
            dimension_semantics=("arbitrary",)
        ),
    )(x, W)


_HALF_ROWS = _ROWS_PER_W // 2
_HALF_GROUPS = _HALF_ROWS // _LANES
_SUB_ROWS = 128
_SUB_GROUPS = _SUB_ROWS // _LANES
_SUBS_PER_HALF = _HALF_ROWS // _SUB_ROWS


def _sc_topk_body(logits_hbm, idx_hbm, val_hbm, blk_v, oi_v, ov_v):
    wid = lax.axis_index("s") * 2 + lax.axis_index("c")

    lane = lax.iota(jnp.int32, _LANES)
    neg_inf = jnp.full((_LANES,), -jnp.inf, jnp.float32)
    _CHUNK = 8
    _NCHUNK = _NUM_EXPERTS // _CHUNK

    def _tourney(pairs):
        while len(pairs) > 1:
            nxt = []
            for j in range(0, len(pairs), 2):
                (lv, li), (hv, hi) = pairs[j], pairs[j + 1]
                gt = hv > lv
                nxt.append((jnp.where(gt, hv, lv), jnp.where(gt, hi, li)))
            pairs = nxt
        return pairs[0]

    for h in range(2):
        pltpu.sync_copy(
            logits_hbm.at[pl.ds(wid * _ROWS_PER_W + h * _HALF_ROWS,
                                _HALF_ROWS), :],
            blk_v,
        )

        def sub_body(s, carry, _h=h):
            sub_first = s * _SUB_ROWS

            @plsc.parallel_loop(0, _SUB_GROUPS, unroll=4)
            def group_body(g):
                rows = sub_first + g * _LANES + lane

                chunk_best = []
                for c in range(_NCHUNK):
                    leaves = []
                    for i in range(_CHUNK):
                        e = c * _CHUNK + i
                        col = jnp.full((_LANES,), e, jnp.int32)
                        v = plsc.load_gather(blk_v, [rows, col])
                        leaves.append((v, col))
                    chunk_best.append(_tourney(leaves))

                sel_vals = []
                sel_idxs = []
                for k in range(_TOP_K):
                    wv, wi = _tourney(list(chunk_best))
                    sel_vals.append(wv)
                    sel_idxs.append(wi)
                    if k + 1 == _TOP_K:
                        break
                    plsc.store_scatter(blk_v, [rows, wi], neg_inf)
                    cb = jnp.bitwise_and(
                        wi, jnp.full((_LANES,), -_CHUNK, jnp.int32))
                    leaves = []
                    for i in range(_CHUNK):
                        ic = jnp.full((_LANES,), i, jnp.int32)
                        v = plsc.load_gather(blk_v, [rows, cb + ic])
                        leaves.append((v, cb + ic))
                    nv, ni = _tourney(leaves)
                    for c in range(_NCHUNK):
                        msk = cb == jnp.full(
                            (_LANES,), c * _CHUNK, jnp.int32)
                        ov, oi = chunk_best[c]
                        chunk_best[c] = (
                            jnp.where(msk, nv, ov),
                            jnp.where(msk, ni, oi),
                        )

                top = sel_vals[0]
                exps = [jnp.exp(v - top) for v in sel_vals]
                total = exps[0]
                for p in exps[1:]:
                    total = total + p

                orow = g * _LANES + lane
                for k in range(_TOP_K):
                    kcol = jnp.full((_LANES,), k, jnp.int32)
                    plsc.store_scatter(oi_v, [orow, kcol], sel_idxs[k])
                    plsc.store_scatter(ov_v, [orow, kcol], exps[k] / total)

            obase = wid * _ROWS_PER_W + _h * _HALF_ROWS + s * _SUB_ROWS
            pltpu.sync_copy(oi_v, idx_hbm.at[pl.ds(obase, _SUB_ROWS), :])
            pltpu.sync_copy(ov_v, val_hbm.at[pl.ds(obase, _SUB_ROWS), :])
            return carry

        lax.fori_loop(0, _SUBS_PER_HALF, sub_body, 0)


def _sc_topk(logits):
    mesh = plsc.VectorSubcoreMesh(core_axis_name="c", subcore_axis_name="s")
    fn = functools.partial(
        pl.kernel,
        mesh=mesh,
        out_type=[
            jax.ShapeDtypeStruct((_NUM_TOKENS, _TOP_K), jnp.int32),
            jax.ShapeDtypeStruct((_NUM_TOKENS, _TOP_K), jnp.float32),
        ],
        scratch_types=[
            pltpu.VMEM((_HALF_ROWS, _NUM_EXPERTS), jnp.float32),
            pltpu.VMEM((_SUB_ROWS, _TOP_K), jnp.int32),
            pltpu.VMEM((_SUB_ROWS, _TOP_K), jnp.float32),
        ],
        compiler_params=pltpu.CompilerParams(needs_layout_passes=False),
    )(_sc_topk_body)
    return fn(logits)


def kernel(x, W):
    logits, probs = _tc_logits_probs(x, W)
    top_k_indices, top_k_probs = _sc_topk(logits)
    return (logits, probs, top_k_indices, top_k_probs)


# speedup vs baseline: 1.0183x; 1.0183x over previous
import functools

import jax
import jax.numpy as jnp
from jax import lax
from jax.experimental import pallas as pl
from jax.experimental.pallas import tpu as pltpu
from jax.experimental.pallas import tpu_sc as plsc

_NUM_TOKENS = 32768
_HIDDEN = 768
_NUM_EXPERTS = 64
_TOP_K = 8

_ROW_BLOCK = 4096

_NUM_WORKERS = 32
_ROWS_PER_W = _NUM_TOKENS // _NUM_WORKERS
_LANES = 16
_GROUPS_PER_W = _ROWS_PER_W // _LANES

_LG_WROWS = _NUM_TOKENS * _NUM_EXPERTS // 128
_LG_WROWS_W = _LG_WROWS // _NUM_WORKERS
_OUT_WROWS = _NUM_TOKENS * _TOP_K // 128
_OUT_WROWS_W = _OUT_WROWS // _NUM_WORKERS


def _tc_body(x_ref, w_ref, logits_ref, probs_ref):
    x = x_ref[...]
    w = w_ref[...]
    logits = lax.dot_general(
        x, w, (((1,), (1,)), ((), ())), preferred_element_type=jnp.float32
    )
    m = jnp.max(logits, axis=1, keepdims=True)
    e = jnp.exp(logits - m)
    probs = e / jnp.sum(e, axis=1, keepdims=True)
    logits_ref[...] = logits
    probs_ref[...] = probs


def _tc_logits_probs(x, W):
    grid = (_NUM_TOKENS // _ROW_BLOCK,)
    out_shape = jax.ShapeDtypeStruct((_NUM_TOKENS, _NUM_EXPERTS), jnp.float32)
    return pl.pallas_call(
        _tc_body,
        grid=grid,
        in_specs=[
            pl.BlockSpec((_ROW_BLOCK, _HIDDEN), lambda i: (i, 0)),
            pl.BlockSpec((_NUM_EXPERTS, _HIDDEN), lambda i: (0, 0)),
        ],
        out_specs=[
            pl.BlockSpec((_ROW_BLOCK, _NUM_EXPERTS), lambda i: (i, 0)),
            pl.BlockSpec((_ROW_BLOCK, _NUM_EXPERTS), lambda i: (i, 0)),
        ],
        out_shape=[out_shape, out_shape],
        compiler_params=pltpu.CompilerParams(
            dimension_semantics=("arbitrary",)
        ),
    )(x, W)


_HALF_ROWS = _ROWS_PER_W // 2
_HALF_GROUPS = _HALF_ROWS // _LANES
_SUB_ROWS = 128
_SUB_GROUPS = _SUB_ROWS // _LANES
_SUBS_PER_HALF = _HALF_ROWS // _SUB_ROWS


def _sc_topk_body(logits_hbm, idx_hbm, val_hbm, blk_v, oi_v, ov_v):
    wid = lax.axis_index("s") * 2 + lax.axis_index("c")

    lane = lax.iota(jnp.int32, _LANES)
    neg_inf = jnp.full((_LANES,), -jnp.inf, jnp.float32)
    _CHUNK = 8
    _NCHUNK = _NUM_EXPERTS // _CHUNK

    def _tourney(pairs):
        while len(pairs) > 1:
            nxt = []
            for j in range(0, len(pairs), 2):
                (lv, li), (hv, hi) = pairs[j], pairs[j + 1]
                gt = hv > lv
                nxt.append((jnp.where(gt, hv, lv), jnp.where(gt, hi, li)))
            pairs = nxt
        return pairs[0]

    for h in range(2):
        pltpu.sync_copy(
            logits_hbm.at[pl.ds(wid * _ROWS_PER_W + h * _HALF_ROWS,
                                _HALF_ROWS), :],
            blk_v,
        )

        def sub_body(s, carry, _h=h):
            sub_first = s * _SUB_ROWS

            @plsc.parallel_loop(0, _SUB_GROUPS, unroll=2)
            def group_body(g):
                rows = sub_first + g * _LANES + lane

                chunk_best = []
                for c in range(_NCHUNK):
                    leaves = []
                    for i in range(_CHUNK):
                        e = c * _CHUNK + i
                        col = jnp.full((_LANES,), e, jnp.int32)
                        v = plsc.load_gather(blk_v, [rows, col])
                        leaves.append((v, col))
                    chunk_best.append(_tourney(leaves))

                sel_vals = []
                sel_idxs = []
                for k in range(_TOP_K):
                    wv, wi = _tourney(list(chunk_best))
                    sel_vals.append(wv)
                    sel_idxs.append(wi)
                    if k + 1 == _TOP_K:
                        break
                    cb = jnp.bitwise_and(
                        wi, jnp.full((_LANES,), -_CHUNK, jnp.int32))
                    leaves = []
                    for i in range(_CHUNK):
                        ic = jnp.full((_LANES,), i, jnp.int32)
                        li = cb + ic
                        v = plsc.load_gather(blk_v, [rows, li])
                        for t in sel_idxs:
                            v = jnp.where(li == t, neg_inf, v)
                        leaves.append((v, li))
                    nv, ni = _tourney(leaves)
                    for c in range(_NCHUNK):
                        msk = cb == jnp.full(
                            (_LANES,), c * _CHUNK, jnp.int32)
                        ov, oi = chunk_best[c]
                        chunk_best[c] = (
                            jnp.where(msk, nv, ov),
                            jnp.where(msk, ni, oi),
                        )

                top = sel_vals[0]
                exps = [jnp.exp(v - top) for v in sel_vals]
                total = exps[0]
                for p in exps[1:]:
                    total = total + p

                orow = g * _LANES + lane
                for k in range(_TOP_K):
                    kcol = jnp.full((_LANES,), k, jnp.int32)
                    plsc.store_scatter(oi_v, [orow, kcol], sel_idxs[k])
                    plsc.store_scatter(ov_v, [orow, kcol], exps[k] / total)

            obase = wid * _ROWS_PER_W + _h * _HALF_ROWS + s * _SUB_ROWS
            pltpu.sync_copy(oi_v, idx_hbm.at[pl.ds(obase, _SUB_ROWS), :])
            pltpu.sync_copy(ov_v, val_hbm.at[pl.ds(obase, _SUB_ROWS), :])
            return carry

        lax.fori_loop(0, _SUBS_PER_HALF, sub_body, 0)


def _sc_topk(logits):
    mesh = plsc.VectorSubcoreMesh(core_axis_name="c", subcore_axis_name="s")
    fn = functools.partial(
        pl.kernel,
        mesh=mesh,
        out_type=[
            jax.ShapeDtypeStruct((_NUM_TOKENS, _TOP_K), jnp.int32),
            jax.ShapeDtypeStruct((_NUM_TOKENS, _TOP_K), jnp.float32),
        ],
        scratch_types=[
            pltpu.VMEM((_HALF_ROWS, _NUM_EXPERTS), jnp.float32),
            pltpu.VMEM((_SUB_ROWS, _TOP_K), jnp.int32),
            pltpu.VMEM((_SUB_ROWS, _TOP_K), jnp.float32),
        ],
        compiler_params=pltpu.CompilerParams(needs_layout_passes=False),
    )(_sc_topk_body)
    return fn(logits)


def kernel(x, W):
    logits, probs = _tc_logits_probs(x, W)
    top_k_indices, top_k_probs = _sc_topk(logits)
    return (logits, probs, top_k_indices, top_k_probs)


# speedup vs baseline: 1.0662x; 1.0470x over previous
import functools

import jax
import jax.numpy as jnp
from jax import lax
from jax.experimental import pallas as pl
from jax.experimental.pallas import tpu as pltpu
from jax.experimental.pallas import tpu_sc as plsc

_NUM_TOKENS = 32768
_HIDDEN = 768
_NUM_EXPERTS = 64
_TOP_K = 8

_ROW_BLOCK = 4096

_NUM_WORKERS = 32
_ROWS_PER_W = _NUM_TOKENS // _NUM_WORKERS
_LANES = 16
_GROUPS_PER_W = _ROWS_PER_W // _LANES

_LG_WROWS = _NUM_TOKENS * _NUM_EXPERTS // 128
_LG_WROWS_W = _LG_WROWS // _NUM_WORKERS
_OUT_WROWS = _NUM_TOKENS * _TOP_K // 128
_OUT_WROWS_W = _OUT_WROWS // _NUM_WORKERS


def _tc_body(x_ref, w_ref, logits_ref, probs_ref):
    x = x_ref[...]
    w = w_ref[...]
    logits = lax.dot_general(
        x, w, (((1,), (1,)), ((), ())), preferred_element_type=jnp.float32
    )
    m = jnp.max(logits, axis=1, keepdims=True)
    e = jnp.exp(logits - m)
    probs = e / jnp.sum(e, axis=1, keepdims=True)
    logits_ref[...] = logits
    probs_ref[...] = probs


def _tc_logits_probs(x, W):
    grid = (_NUM_TOKENS // _ROW_BLOCK,)
    out_shape = jax.ShapeDtypeStruct((_NUM_TOKENS, _NUM_EXPERTS), jnp.float32)
    return pl.pallas_call(
        _tc_body,
        grid=grid,
        in_specs=[
            pl.BlockSpec((_ROW_BLOCK, _HIDDEN), lambda i: (i, 0)),
            pl.BlockSpec((_NUM_EXPERTS, _HIDDEN), lambda i: (0, 0)),
        ],
        out_specs=[
            pl.BlockSpec((_ROW_BLOCK, _NUM_EXPERTS), lambda i: (i, 0)),
            pl.BlockSpec((_ROW_BLOCK, _NUM_EXPERTS), lambda i: (i, 0)),
        ],
        out_shape=[out_shape, out_shape],
        compiler_params=pltpu.CompilerParams(
            dimension_semantics=("arbitrary",)
        ),
    )(x, W)


_HALF_ROWS = _ROWS_PER_W // 2
_HALF_GROUPS = _HALF_ROWS // _LANES
_SUB_ROWS = 128
_SUB_GROUPS = _SUB_ROWS // _LANES
_SUBS_PER_HALF = _HALF_ROWS // _SUB_ROWS


def _sc_topk_body(logits_hbm, idx_hbm, val_hbm, blk_v, oi_v, ov_v):
    wid = lax.axis_index("s") * 2 + lax.axis_index("c")

    lane = lax.iota(jnp.int32, _LANES)
    neg_inf = jnp.full((_LANES,), -jnp.inf, jnp.float32)
    _CHUNK = 8
    _NCHUNK = _NUM_EXPERTS // _CHUNK

    def _tourney(pairs):
        while len(pairs) > 1:
            nxt = []
            for j in range(0, len(pairs), 2):
                (lv, li), (hv, hi) = pairs[j], pairs[j + 1]
                gt = hv > lv
                nxt.append((jnp.where(gt, hv, lv), jnp.where(gt, hi, li)))
            pairs = nxt
        return pairs[0]

    for h in range(2):
        pltpu.sync_copy(
            logits_hbm.at[pl.ds(wid * _ROWS_PER_W + h * _HALF_ROWS,
                                _HALF_ROWS), :],
            blk_v,
        )

        def sub_body(s, carry, _h=h):
            sub_first = s * _SUB_ROWS

            @plsc.parallel_loop(0, _SUB_GROUPS, unroll=1)
            def group_body(g):
                rows = sub_first + g * _LANES + lane

                chunk_best = []
                for c in range(_NCHUNK):
                    leaves = []
                    for i in range(_CHUNK):
                        e = c * _CHUNK + i
                        col = jnp.full((_LANES,), e, jnp.int32)
                        v = plsc.load_gather(blk_v, [rows, col])
                        leaves.append((v, col))
                    chunk_best.append(_tourney(leaves))

                sel_vals = []
                sel_idxs = []
                for k in range(_TOP_K):
                    wv, wi = _tourney(list(chunk_best))
                    sel_vals.append(wv)
                    sel_idxs.append(wi)
                    if k + 1 == _TOP_K:
                        break
                    plsc.store_scatter(blk_v, [rows, wi], neg_inf)
                    cb = jnp.bitwise_and(
                        wi, jnp.full((_LANES,), -_CHUNK, jnp.int32))
                    leaves = []
                    for i in range(_CHUNK):
                        ic = jnp.full((_LANES,), i, jnp.int32)
                        v = plsc.load_gather(blk_v, [rows, cb + ic])
                        leaves.append((v, cb + ic))
                    nv, ni = _tourney(leaves)
                    for c in range(_NCHUNK):
                        msk = cb == jnp.full(
                            (_LANES,), c * _CHUNK, jnp.int32)
                        ov, oi = chunk_best[c]
                        chunk_best[c] = (
                            jnp.where(msk, nv, ov),
                            jnp.where(msk, ni, oi),
                        )

                top = sel_vals[0]
                exps = [jnp.exp(v - top) for v in sel_vals]
                total = exps[0]
                for p in exps[1:]:
                    total = total + p

                orow = g * _LANES + lane
                for k in range(_TOP_K):
                    kcol = jnp.full((_LANES,), k, jnp.int32)
                    plsc.store_scatter(oi_v, [orow, kcol], sel_idxs[k])
                    plsc.store_scatter(ov_v, [orow, kcol], exps[k] / total)

            obase = wid * _ROWS_PER_W + _h * _HALF_ROWS + s * _SUB_ROWS
            pltpu.sync_copy(oi_v, idx_hbm.at[pl.ds(obase, _SUB_ROWS), :])
            pltpu.sync_copy(ov_v, val_hbm.at[pl.ds(obase, _SUB_ROWS), :])
            return carry

        lax.fori_loop(0, _SUBS_PER_HALF, sub_body, 0)


def _sc_topk(logits):
    mesh = plsc.VectorSubcoreMesh(core_axis_name="c", subcore_axis_name="s")
    fn = functools.partial(
        pl.kernel,
        mesh=mesh,
        out_type=[
            jax.ShapeDtypeStruct((_NUM_TOKENS, _TOP_K), jnp.int32),
            jax.ShapeDtypeStruct((_NUM_TOKENS, _TOP_K), jnp.float32),
        ],
        scratch_types=[
            pltpu.VMEM((_HALF_ROWS, _NUM_EXPERTS), jnp.float32),
            pltpu.VMEM((_SUB_ROWS, _TOP_K), jnp.int32),
            pltpu.VMEM((_SUB_ROWS, _TOP_K), jnp.float32),
        ],
        compiler_params=pltpu.CompilerParams(needs_layout_passes=False),
    )(_sc_topk_body)
    return fn(logits)


def kernel(x, W):
    logits, probs = _tc_logits_probs(x, W)
    top_k_indices, top_k_probs = _sc_topk(logits)
    return (logits, probs, top_k_indices, top_k_probs)


# speedup vs baseline: 1.0679x; 1.0016x over previous
import functools

import jax
import jax.numpy as jnp
from jax import lax
from jax.experimental import pallas as pl
from jax.experimental.pallas import tpu as pltpu
from jax.experimental.pallas import tpu_sc as plsc

_NUM_TOKENS = 32768
_HIDDEN = 768
_NUM_EXPERTS = 64
_TOP_K = 8

_ROW_BLOCK = 4096

_NUM_WORKERS = 32
_ROWS_PER_W = _NUM_TOKENS // _NUM_WORKERS
_LANES = 16
_GROUPS_PER_W = _ROWS_PER_W // _LANES


def _tc_body(x_ref, w_ref, logits_ref, probs_ref):
    x = x_ref[...]
    w = w_ref[...]
    logits = lax.dot_general(
        x, w, (((1,), (1,)), ((), ())), preferred_element_type=jnp.float32
    )
    m = jnp.max(logits, axis=1, keepdims=True)
    e = jnp.exp(logits - m)
    probs = e / jnp.sum(e, axis=1, keepdims=True)
    logits_ref[...] = logits
    probs_ref[...] = probs


def _tc_logits_probs(x, W):
    grid = (_NUM_TOKENS // _ROW_BLOCK,)
    out_shape = jax.ShapeDtypeStruct((_NUM_TOKENS, _NUM_EXPERTS), jnp.float32)
    return pl.pallas_call(
        _tc_body,
        grid=grid,
        in_specs=[
            pl.BlockSpec((_ROW_BLOCK, _HIDDEN), lambda i: (i, 0)),
            pl.BlockSpec((_NUM_EXPERTS, _HIDDEN), lambda i: (0, 0)),
        ],
        out_specs=[
            pl.BlockSpec((_ROW_BLOCK, _NUM_EXPERTS), lambda i: (i, 0)),
            pl.BlockSpec((_ROW_BLOCK, _NUM_EXPERTS), lambda i: (i, 0)),
        ],
        out_shape=[out_shape, out_shape],
        compiler_params=pltpu.CompilerParams(
            dimension_semantics=("arbitrary",)
        ),
    )(x, W)


_HALF_ROWS = _ROWS_PER_W // 2
_HALF_GROUPS = _HALF_ROWS // _LANES
_SUB_ROWS = 128
_SUB_GROUPS = _SUB_ROWS // _LANES
_SUBS_PER_HALF = _HALF_ROWS // _SUB_ROWS


def _sc_topk_body(logits_hbm, idx_hbm, val_hbm, blk_v, oi_v, ov_v):
    wid = lax.axis_index("s") * 2 + lax.axis_index("c")

    lane = lax.iota(jnp.int32, _LANES)
    neg_inf = jnp.full((_LANES,), -jnp.inf, jnp.float32)
    _CHUNK = 8
    _NCHUNK = _NUM_EXPERTS // _CHUNK

    def _tourney(pairs):
        while len(pairs) > 1:
            nxt = []
            for j in range(0, len(pairs), 2):
                (lv, li), (hv, hi) = pairs[j], pairs[j + 1]
                gt = hv > lv
                nxt.append((jnp.where(gt, hv, lv), jnp.where(gt, hi, li)))
            pairs = nxt
        return pairs[0]

    for h in range(2):
        pltpu.sync_copy(
            logits_hbm.at[pl.ds(wid * _ROWS_PER_W + h * _HALF_ROWS,
                                _HALF_ROWS), :],
            blk_v,
        )

        def sub_body(s, carry, _h=h):
            sub_first = s * _SUB_ROWS

            @plsc.parallel_loop(0, _SUB_GROUPS, unroll=1)
            def group_body(g):
                rows = sub_first + g * _LANES + lane

                chunk_best = []
                for c in range(_NCHUNK):
                    leaves = []
                    for i in range(_CHUNK):
                        e = c * _CHUNK + i
                        col = jnp.full((_LANES,), e, jnp.int32)
                        v = plsc.load_gather(blk_v, [rows, col])
                        leaves.append((v, col))
                    chunk_best.append(_tourney(leaves))

                sel_vals = []
                sel_idxs = []
                for k in range(_TOP_K):
                    wv, wi = _tourney(list(chunk_best))
                    sel_vals.append(wv)
                    sel_idxs.append(wi)
                    if k + 1 == _TOP_K:
                        break
                    plsc.store_scatter(blk_v, [rows, wi], neg_inf)
                    cb = jnp.bitwise_and(
                        wi, jnp.full((_LANES,), -_CHUNK, jnp.int32))
                    leaves = []
                    for i in range(_CHUNK):
                        ic = jnp.full((_LANES,), i, jnp.int32)
                        v = plsc.load_gather(blk_v, [rows, cb + ic])
                        leaves.append((v, cb + ic))
                    nv, ni = _tourney(leaves)
                    for c in range(_NCHUNK):
                        msk = cb == jnp.full(
                            (_LANES,), c * _CHUNK, jnp.int32)
                        ov, oi = chunk_best[c]
                        chunk_best[c] = (
                            jnp.where(msk, nv, ov),
                            jnp.where(msk, ni, oi),
                        )

                top = sel_vals[0]
                exps = [jnp.exp(v - top) for v in sel_vals]
                total = exps[0]
                for p in exps[1:]:
                    total = total + p

                orow = g * _LANES + lane
                for k in range(_TOP_K):
                    kcol = jnp.full((_LANES,), k, jnp.int32)
                    plsc.store_scatter(oi_v, [orow, kcol], sel_idxs[k])
                    plsc.store_scatter(ov_v, [orow, kcol], exps[k] / total)

            obase = wid * _ROWS_PER_W + _h * _HALF_ROWS + s * _SUB_ROWS
            pltpu.sync_copy(oi_v, idx_hbm.at[pl.ds(obase, _SUB_ROWS), :])
            pltpu.sync_copy(ov_v, val_hbm.at[pl.ds(obase, _SUB_ROWS), :])
            return carry

        lax.fori_loop(0, _SUBS_PER_HALF, sub_body, 0)


def _sc_topk(logits):
    mesh = plsc.VectorSubcoreMesh(core_axis_name="c", subcore_axis_name="s")
    fn = functools.partial(
        pl.kernel,
        mesh=mesh,
        out_type=[
            jax.ShapeDtypeStruct((_NUM_TOKENS, _TOP_K), jnp.int32),
            jax.ShapeDtypeStruct((_NUM_TOKENS, _TOP_K), jnp.float32),
        ],
        scratch_types=[
            pltpu.VMEM((_HALF_ROWS, _NUM_EXPERTS), jnp.float32),
            pltpu.VMEM((_SUB_ROWS, _TOP_K), jnp.int32),
            pltpu.VMEM((_SUB_ROWS, _TOP_K), jnp.float32),
        ],
        compiler_params=pltpu.CompilerParams(needs_layout_passes=False),
    )(_sc_topk_body)
    return fn(logits)


def kernel(x, W):
    logits, probs = _tc_logits_probs(x, W)
    top_k_indices, top_k_probs = _sc_topk(logits)
    return (logits, probs, top_k_indices, top_k_probs)
